# baseline (speedup 1.0000x reference)
import functools

import jax
import jax.numpy as jnp
from jax import lax
from jax.experimental import pallas as pl
from jax.experimental.pallas import tpu as pltpu
from jax.experimental.pallas import tpu_sc as plsc

N = 100000
E = 6400000
CLAMP = 10.0

NSC = 2
NTILE = 16
NWORK = NSC * NTILE
NPAD = 100352
SLICE1 = NPAD // NTILE
SLICE2 = NPAD // NWORK
EPT = E // NWORK
W = 2000
NWIN = EPT // W

_mesh = plsc.VectorSubcoreMesh(core_axis_name="c", subcore_axis_name="s")


@functools.partial(
    pl.kernel,
    out_type=jax.ShapeDtypeStruct((NWORK * NPAD,), jnp.float32),
    mesh=_mesh,
    compiler_params=pltpu.CompilerParams(needs_layout_passes=False),
    scratch_types=[
        pltpu.VMEM((NPAD,), jnp.float32),
        [pltpu.VMEM((W,), jnp.int32)] * 2,
        [pltpu.VMEM((W,), jnp.int32)] * 2,
        [pltpu.VMEM((W,), jnp.float32)] * 2,
        [pltpu.VMEM((W,), jnp.float32)] * 2,
        pltpu.VMEM_SHARED((NPAD,), jnp.float32),
        [pltpu.SemaphoreType.DMA] * 2,
        [pltpu.SemaphoreType.DMA] * 2,
        pltpu.SemaphoreType.DMA,
    ],
)
def _scatter_phase(v_hbm, src_hbm, dst_hbm, w_hbm, out_hbm,
                   acc, srcb, dstb, wb, valb, vsh, lsem, gsem, psem):
    cid = lax.axis_index("c")
    sid = lax.axis_index("s")
    gwid = cid * NTILE + sid
    nbase = sid * SLICE1
    ebase = gwid * EPT

    def _issue_loads(g, b):
        off = ebase + g * W
        pltpu.async_copy(src_hbm.at[pl.ds(off, W)], srcb[b], lsem[b])
        pltpu.async_copy(dst_hbm.at[pl.ds(off, W)], dstb[b], lsem[b])
        pltpu.async_copy(w_hbm.at[pl.ds(off, W)], wb[b], lsem[b])

    def _wait_loads(b):
        pltpu.make_async_copy(src_hbm.at[pl.ds(0, W)], srcb[b], lsem[b]).wait()
        pltpu.make_async_copy(dst_hbm.at[pl.ds(0, W)], dstb[b], lsem[b]).wait()
        pltpu.make_async_copy(w_hbm.at[pl.ds(0, W)], wb[b], lsem[b]).wait()

    def _issue_gather(b):
        pltpu.async_copy(vsh.at[srcb[b]], valb[b], gsem[b])

    def _wait_gather(b):
        pltpu.make_async_copy(vsh.at[srcb[b]], valb[b], gsem[b]).wait()

    def _zero(j, carry):
        acc[pl.ds(j * 16, 16)] = jnp.zeros((16,), jnp.float32)
        return carry
    lax.fori_loop(0, NPAD // 16, _zero, 0, unroll=8)

    pltpu.sync_copy(v_hbm.at[pl.ds(nbase, SLICE1)],
                    vsh.at[pl.ds(nbase, SLICE1)])
    _issue_loads(0, 0)
    _wait_loads(0)
    plsc.subcore_barrier()
    _issue_gather(0)
    _issue_loads(1, 1)

    def _compute(b):
        def body(j, carry):
            sl = pl.ds(j * 16, 16)
            x = valb[b][sl]
            m = jnp.minimum(jnp.maximum(x, 0.0), CLAMP) * wb[b][sl]
            plsc.addupdate_scatter(acc, [dstb[b][sl]], m)
            return carry
        lax.fori_loop(0, W // 16, body, 0, unroll=8)

    def _pair(m, carry):
        _wait_gather(0)
        _wait_loads(1)
        _issue_gather(1)
        _compute(0)

        @pl.when(m < NWIN // 2 - 1)
        def _():
            _issue_loads(2 * m + 2, 0)

        _wait_gather(1)

        @pl.when(m < NWIN // 2 - 1)
        def _():
            _wait_loads(0)
            _issue_gather(0)
        _compute(1)

        @pl.when(m < NWIN // 2 - 1)
        def _():
            _issue_loads(2 * m + 3, 1)
        return carry

    lax.fori_loop(0, NWIN // 2, _pair, 0)

    for j in range(NWORK):
        pltpu.async_copy(
            acc.at[pl.ds(j * SLICE2, SLICE2)],
            out_hbm.at[pl.ds(j * (NWORK * SLICE2) + gwid * SLICE2, SLICE2)],
            psem)
    for j in range(NWORK):
        pltpu.make_async_copy(
            acc.at[pl.ds(0, SLICE2)],
            out_hbm.at[pl.ds(0, SLICE2)],
            psem).wait()


@functools.partial(
    pl.kernel,
    out_type=jax.ShapeDtypeStruct((NPAD,), jnp.float32),
    mesh=_mesh,
    compiler_params=pltpu.CompilerParams(needs_layout_passes=False),
    scratch_types=[
        pltpu.VMEM((NWORK * SLICE2,), jnp.float32),
        pltpu.VMEM((SLICE2,), jnp.float32),
        pltpu.VMEM((SLICE2,), jnp.float32),
        pltpu.VMEM((SLICE2,), jnp.float32),
        pltpu.VMEM((SLICE2,), jnp.float32),
        pltpu.VMEM((SLICE2,), jnp.float32),
    ],
)
def _combine_phase(part_hbm, v_hbm, tau_hbm, stim_hbm, bias_hbm, out_hbm,
                   pbuf, vb, tb, sb, bb, ob):
    cid = lax.axis_index("c")
    sid = lax.axis_index("s")
    gwid = cid * NTILE + sid
    nb = gwid * SLICE2
    pltpu.sync_copy(part_hbm.at[pl.ds(gwid * (NWORK * SLICE2), NWORK * SLICE2)],
                    pbuf)
    pltpu.sync_copy(v_hbm.at[pl.ds(nb, SLICE2)], vb)
    pltpu.sync_copy(tau_hbm.at[pl.ds(nb, SLICE2)], tb)
    pltpu.sync_copy(stim_hbm.at[pl.ds(nb, SLICE2)], sb)
    pltpu.sync_copy(bias_hbm.at[pl.ds(nb, SLICE2)], bb)

    def _compute(j, carry):
        sl = pl.ds(j * 16, 16)
        s = pbuf[sl]
        for r in range(1, NWORK):
            s = s + pbuf[pl.ds(r * SLICE2 + j * 16, 16)]
        vc = jnp.minimum(jnp.maximum(vb[sl], -CLAMP), CLAMP)
        ob[sl] = (s + sb[sl] + bb[sl] - vc) / tb[sl]
        return carry
    lax.fori_loop(0, SLICE2 // 16, _compute, 0, unroll=2)
    pltpu.sync_copy(ob, out_hbm.at[pl.ds(nb, SLICE2)])


def kernel(t, v, edge_index, w_edge, tau, stimulus, bias):
    pad = NPAD - N
    vp = jnp.pad(v, (0, pad))
    taup = jnp.pad(tau, (0, pad), constant_values=1.0)
    stimp = jnp.pad(stimulus, (0, pad))
    biasp = jnp.pad(bias, (0, pad))
    src = edge_index[0]
    dst = edge_index[1]
    partial = _scatter_phase(vp, src, dst, w_edge)
    dvp = _combine_phase(partial, vp, taup, stimp, biasp)
    return dvp[:N]


# speedup vs baseline: 1.0715x; 1.0715x over previous
import functools

import jax
import jax.numpy as jnp
from jax import lax
from jax.experimental import pallas as pl
from jax.experimental.pallas import tpu as pltpu
from jax.experimental.pallas import tpu_sc as plsc

N = 100000
E = 6400000
CLAMP = 10.0

NSC = 2
NTILE = 16
NWORK = NSC * NTILE
NPAD = 100352
SLICE1 = NPAD // NTILE
SLICE2 = NPAD // NWORK
EPT = E // NWORK
W = 2000
NWIN = EPT // W

_mesh = plsc.VectorSubcoreMesh(core_axis_name="c", subcore_axis_name="s")


@functools.partial(
    pl.kernel,
    out_type=jax.ShapeDtypeStruct((NWORK * NPAD,), jnp.float32),
    mesh=_mesh,
    compiler_params=pltpu.CompilerParams(needs_layout_passes=False),
    scratch_types=[
        pltpu.VMEM((NPAD,), jnp.float32),
        [pltpu.VMEM((W,), jnp.int32)] * 2,
        [pltpu.VMEM((W,), jnp.int32)] * 2,
        [pltpu.VMEM((W,), jnp.float32)] * 2,
        [pltpu.VMEM((W,), jnp.float32)] * 2,
        pltpu.VMEM_SHARED((NPAD,), jnp.float32),
        [pltpu.SemaphoreType.DMA] * 2,
        [pltpu.SemaphoreType.DMA] * 2,
        pltpu.SemaphoreType.DMA,
    ],
)
def _scatter_phase(v_hbm, src_hbm, dst_hbm, w_hbm, out_hbm,
                   acc, srcb, dstb, wb, valb, vsh, lsem, gsem, psem):
    cid = lax.axis_index("c")
    sid = lax.axis_index("s")
    gwid = cid * NTILE + sid
    nbase = sid * SLICE1
    ebase = gwid * EPT

    def _issue_loads(g, b):
        off = ebase + g * W
        pltpu.async_copy(src_hbm.at[pl.ds(off, W)], srcb[b], lsem[b])
        pltpu.async_copy(dst_hbm.at[pl.ds(off, W)], dstb[b], lsem[b])
        pltpu.async_copy(w_hbm.at[pl.ds(off, W)], wb[b], lsem[b])

    def _wait_loads(b):
        pltpu.make_async_copy(src_hbm.at[pl.ds(0, W)], srcb[b], lsem[b]).wait()
        pltpu.make_async_copy(dst_hbm.at[pl.ds(0, W)], dstb[b], lsem[b]).wait()
        pltpu.make_async_copy(w_hbm.at[pl.ds(0, W)], wb[b], lsem[b]).wait()

    NG = 5
    WG = W // NG

    def _issue_gather(b):
        for q in range(NG):
            pltpu.async_copy(vsh.at[srcb[b].at[pl.ds(q * WG, WG)]],
                             valb[b].at[pl.ds(q * WG, WG)], gsem[b])

    def _wait_gather(b):
        for q in range(NG):
            pltpu.make_async_copy(vsh.at[srcb[b].at[pl.ds(q * WG, WG)]],
                                  valb[b].at[pl.ds(q * WG, WG)],
                                  gsem[b]).wait()

    def _zero(j, carry):
        acc[pl.ds(j * 16, 16)] = jnp.zeros((16,), jnp.float32)
        return carry
    lax.fori_loop(0, NPAD // 16, _zero, 0, unroll=8)

    pltpu.sync_copy(v_hbm.at[pl.ds(nbase, SLICE1)],
                    vsh.at[pl.ds(nbase, SLICE1)])
    _issue_loads(0, 0)
    _wait_loads(0)
    plsc.subcore_barrier()
    _issue_gather(0)
    _issue_loads(1, 1)

    def _compute(b):
        def body(j, carry):
            sl = pl.ds(j * 16, 16)
            x = valb[b][sl]
            m = jnp.minimum(jnp.maximum(x, 0.0), CLAMP) * wb[b][sl]
            plsc.addupdate_scatter(acc, [dstb[b][sl]], m)
            return carry
        lax.fori_loop(0, W // 16, body, 0, unroll=8)

    def _pair(m, carry):
        _wait_gather(0)
        _compute(0)
        _wait_loads(1)
        _issue_gather(1)

        @pl.when(m < NWIN // 2 - 1)
        def _():
            _issue_loads(2 * m + 2, 0)

        _wait_gather(1)
        _compute(1)

        @pl.when(m < NWIN // 2 - 1)
        def _():
            _wait_loads(0)
            _issue_gather(0)
            _issue_loads(2 * m + 3, 1)
        return carry

    lax.fori_loop(0, NWIN // 2, _pair, 0)

    for j in range(NWORK):
        pltpu.async_copy(
            acc.at[pl.ds(j * SLICE2, SLICE2)],
            out_hbm.at[pl.ds(j * (NWORK * SLICE2) + gwid * SLICE2, SLICE2)],
            psem)
    for j in range(NWORK):
        pltpu.make_async_copy(
            acc.at[pl.ds(0, SLICE2)],
            out_hbm.at[pl.ds(0, SLICE2)],
            psem).wait()


@functools.partial(
    pl.kernel,
    out_type=jax.ShapeDtypeStruct((NPAD,), jnp.float32),
    mesh=_mesh,
    compiler_params=pltpu.CompilerParams(needs_layout_passes=False),
    scratch_types=[
        pltpu.VMEM((NWORK * SLICE2,), jnp.float32),
        pltpu.VMEM((SLICE2,), jnp.float32),
        pltpu.VMEM((SLICE2,), jnp.float32),
        pltpu.VMEM((SLICE2,), jnp.float32),
        pltpu.VMEM((SLICE2,), jnp.float32),
        pltpu.VMEM((SLICE2,), jnp.float32),
    ],
)
def _combine_phase(part_hbm, v_hbm, tau_hbm, stim_hbm, bias_hbm, out_hbm,
                   pbuf, vb, tb, sb, bb, ob):
    cid = lax.axis_index("c")
    sid = lax.axis_index("s")
    gwid = cid * NTILE + sid
    nb = gwid * SLICE2
    pltpu.sync_copy(part_hbm.at[pl.ds(gwid * (NWORK * SLICE2), NWORK * SLICE2)],
                    pbuf)
    pltpu.sync_copy(v_hbm.at[pl.ds(nb, SLICE2)], vb)
    pltpu.sync_copy(tau_hbm.at[pl.ds(nb, SLICE2)], tb)
    pltpu.sync_copy(stim_hbm.at[pl.ds(nb, SLICE2)], sb)
    pltpu.sync_copy(bias_hbm.at[pl.ds(nb, SLICE2)], bb)

    def _compute(j, carry):
        sl = pl.ds(j * 16, 16)
        s = pbuf[sl]
        for r in range(1, NWORK):
            s = s + pbuf[pl.ds(r * SLICE2 + j * 16, 16)]
        vc = jnp.minimum(jnp.maximum(vb[sl], -CLAMP), CLAMP)
        ob[sl] = (s + sb[sl] + bb[sl] - vc) / tb[sl]
        return carry
    lax.fori_loop(0, SLICE2 // 16, _compute, 0, unroll=2)
    pltpu.sync_copy(ob, out_hbm.at[pl.ds(nb, SLICE2)])


def kernel(t, v, edge_index, w_edge, tau, stimulus, bias):
    pad = NPAD - N
    vp = jnp.pad(v, (0, pad))
    taup = jnp.pad(tau, (0, pad), constant_values=1.0)
    stimp = jnp.pad(stimulus, (0, pad))
    biasp = jnp.pad(bias, (0, pad))
    src = edge_index[0]
    dst = edge_index[1]
    partial = _scatter_phase(vp, src, dst, w_edge)
    dvp = _combine_phase(partial, vp, taup, stimp, biasp)
    return dvp[:N]


# speedup vs baseline: 1.1136x; 1.0393x over previous
import functools

import jax
import jax.numpy as jnp
from jax import lax
from jax.experimental import pallas as pl
from jax.experimental.pallas import tpu as pltpu
from jax.experimental.pallas import tpu_sc as plsc

N = 100000
E = 6400000
CLAMP = 10.0

NSC = 2
NTILE = 16
NWORK = NSC * NTILE
NPAD = 100352
SLICE1 = NPAD // NTILE
SLICE2 = NPAD // NWORK
EPT = E // NWORK
W = 2000
NWIN = EPT // W

_mesh = plsc.VectorSubcoreMesh(core_axis_name="c", subcore_axis_name="s")


@functools.partial(
    pl.kernel,
    out_type=jax.ShapeDtypeStruct((NWORK * NPAD,), jnp.float32),
    mesh=_mesh,
    compiler_params=pltpu.CompilerParams(needs_layout_passes=False),
    scratch_types=[
        pltpu.VMEM((NPAD,), jnp.float32),
        [pltpu.VMEM((W,), jnp.int32)] * 2,
        [pltpu.VMEM((W,), jnp.int32)] * 2,
        [pltpu.VMEM((W,), jnp.float32)] * 2,
        [pltpu.VMEM((W,), jnp.float32)] * 2,
        pltpu.VMEM_SHARED((NPAD,), jnp.float32),
        [pltpu.SemaphoreType.DMA] * 2,
        [pltpu.SemaphoreType.DMA] * 2,
        pltpu.SemaphoreType.DMA,
    ],
)
def _scatter_phase(v_hbm, src_hbm, dst_hbm, w_hbm, out_hbm,
                   acc, srcb, dstb, wb, valb, vsh, lsem, gsem, psem):
    cid = lax.axis_index("c")
    sid = lax.axis_index("s")
    gwid = cid * NTILE + sid
    nbase = sid * SLICE1
    ebase = gwid * EPT

    def _issue_loads(g, b):
        off = ebase + g * W
        pltpu.async_copy(src_hbm.at[pl.ds(off, W)], srcb[b], lsem[b])
        pltpu.async_copy(dst_hbm.at[pl.ds(off, W)], dstb[b], lsem[b])
        pltpu.async_copy(w_hbm.at[pl.ds(off, W)], wb[b], lsem[b])

    def _wait_loads(b):
        pltpu.make_async_copy(src_hbm.at[pl.ds(0, W)], srcb[b], lsem[b]).wait()
        pltpu.make_async_copy(dst_hbm.at[pl.ds(0, W)], dstb[b], lsem[b]).wait()
        pltpu.make_async_copy(w_hbm.at[pl.ds(0, W)], wb[b], lsem[b]).wait()

    NG = 5
    WG = W // NG

    def _issue_gather(b):
        for q in range(NG):
            pltpu.async_copy(vsh.at[srcb[b].at[pl.ds(q * WG, WG)]],
                             valb[b].at[pl.ds(q * WG, WG)], gsem[b])

    def _wait_gather(b):
        for q in range(NG):
            pltpu.make_async_copy(vsh.at[srcb[b].at[pl.ds(q * WG, WG)]],
                                  valb[b].at[pl.ds(q * WG, WG)],
                                  gsem[b]).wait()

    def _zero(j, carry):
        acc[pl.ds(j * 16, 16)] = jnp.zeros((16,), jnp.float32)
        return carry
    lax.fori_loop(0, NPAD // 16, _zero, 0, unroll=8)

    pltpu.sync_copy(v_hbm.at[pl.ds(nbase, SLICE1)],
                    vsh.at[pl.ds(nbase, SLICE1)])
    _issue_loads(0, 0)
    _wait_loads(0)
    plsc.subcore_barrier()
    _issue_gather(0)
    _issue_loads(1, 1)

    def _compute(b):
        def body(j, carry):
            sl = pl.ds(j * 16, 16)
            x = valb[b][sl]
            m = jnp.minimum(jnp.maximum(x, 0.0), CLAMP) * wb[b][sl]
            plsc.addupdate_scatter(acc, [dstb[b][sl]], m)
            return carry
        lax.fori_loop(0, W // 16, body, 0, unroll=8)

    def _pair(m, carry):
        _wait_gather(0)
        _compute(0)
        _wait_loads(1)
        _issue_gather(1)

        @pl.when(m < NWIN // 2 - 1)
        def _():
            _issue_loads(2 * m + 2, 0)

        _wait_gather(1)
        _compute(1)

        @pl.when(m < NWIN // 2 - 1)
        def _():
            _wait_loads(0)
            _issue_gather(0)
            _issue_loads(2 * m + 3, 1)
        return carry

    lax.fori_loop(0, NWIN // 2, _pair, 0)

    pltpu.async_copy(acc, out_hbm.at[pl.ds(gwid * NPAD, NPAD)], psem)
    pltpu.make_async_copy(acc, out_hbm.at[pl.ds(gwid * NPAD, NPAD)],
                          psem).wait()


_ROWS = NPAD // 128


def _combine_body(p_ref, v_ref, tau_ref, stim_ref, bias_ref, out_ref):
    agg = jnp.sum(p_ref[...], axis=0)
    vc = jnp.clip(v_ref[...], -CLAMP, CLAMP)
    out_ref[...] = (agg + stim_ref[...] + bias_ref[...] - vc) / tau_ref[...]


def _combine_phase(part, vp, taup, stimp, biasp):
    return pl.pallas_call(
        _combine_body,
        out_shape=jax.ShapeDtypeStruct((_ROWS, 128), jnp.float32),
    )(part.reshape(NWORK, _ROWS, 128), vp.reshape(_ROWS, 128),
      taup.reshape(_ROWS, 128), stimp.reshape(_ROWS, 128),
      biasp.reshape(_ROWS, 128))


def kernel(t, v, edge_index, w_edge, tau, stimulus, bias):
    pad = NPAD - N
    vp = jnp.pad(v, (0, pad))
    taup = jnp.pad(tau, (0, pad), constant_values=1.0)
    stimp = jnp.pad(stimulus, (0, pad))
    biasp = jnp.pad(bias, (0, pad))
    src = edge_index[0]
    dst = edge_index[1]
    partial = _scatter_phase(vp, src, dst, w_edge)
    dvp = _combine_phase(partial, vp, taup, stimp, biasp)
    return dvp.reshape(NPAD)[:N]
